# Initial kernel scaffold; baseline (speedup 1.0000x reference)
import jax
import jax.numpy as jnp
from jax.experimental import pallas as pl


def kernel(x, edge_index, W1, W2, Wd):
    raise NotImplementedError("write your pallas kernel here")


# speedup vs baseline: 15.3427x; 15.3427x over previous
import functools

import jax
import jax.numpy as jnp
from jax import lax
from jax.experimental import pallas as pl
from jax.experimental.pallas import tpu as pltpu
from jax.experimental.pallas import tpu_sc as plsc

N = 10000
E = 320000
D = 128
OUT = 51
OUTP = 64
NPAD = 10240
NC, NS = 2, 16
NT = NC * NS
C = 80
CPT = E // (NT * C)
RPT = NPAD // NS
ZROWS = 128
DSEG = NPAD // NS
RB = 1024
GRID = (N + RB - 1) // RB

_mesh = plsc.VectorSubcoreMesh(
    core_axis_name="c", subcore_axis_name="s", num_cores=NC, num_subcores=NS
)

_Z16 = functools.partial(jnp.zeros, (16,), jnp.float32)


@functools.partial(
    pl.kernel,
    out_type=jax.ShapeDtypeStruct((NC, NPAD), jnp.float32),
    mesh=_mesh,
    scratch_types=[
        pltpu.VMEM((CPT, C), jnp.int32),
        pltpu.VMEM((C,), jnp.float32),
        pltpu.VMEM((DSEG,), jnp.float32),
        pltpu.VMEM_SHARED((NPAD,), jnp.float32),
    ],
)
def _degree_kernel(dst_hbm, out_hbm, dst_v, ones_v, zbuf_v, deg_sh):
    cid = lax.axis_index("c")
    sid = lax.axis_index("s")
    tid = cid * NS + sid

    pltpu.sync_copy(dst_hbm.at[tid], dst_v)

    def _zfill(i, _):
        zbuf_v[pl.ds(i * 16, 16)] = _Z16()
        return 0

    lax.fori_loop(0, DSEG // 16, _zfill, 0)

    def _ofill(i, _):
        ones_v[pl.ds(i * 16, 16)] = jnp.ones((16,), jnp.float32)
        return 0

    lax.fori_loop(0, C // 16, _ofill, 0)

    pltpu.sync_copy(zbuf_v, deg_sh.at[pl.ds(sid * DSEG, DSEG)])
    plsc.subcore_barrier()

    def _scat(r, _):
        pltpu.sync_copy(ones_v, deg_sh.at[dst_v.at[r]], add=True)
        return 0

    lax.fori_loop(0, CPT, _scat, 0)
    plsc.subcore_barrier()

    pltpu.sync_copy(
        deg_sh.at[pl.ds(sid * DSEG, DSEG)],
        out_hbm.at[cid, pl.ds(sid * DSEG, DSEG)],
    )


HROWS = NPAD // 2
TRASH = 128
AROWS = HROWS + TRASH
ZPT = AROWS // NS
WPT = HROWS // NS


@functools.partial(
    pl.kernel,
    out_type=jax.ShapeDtypeStruct((NC, 2, HROWS, D), jnp.float32),
    mesh=_mesh,
    scratch_types=[
        pltpu.VMEM((CPT, C), jnp.int32),
        pltpu.VMEM((CPT, C), jnp.int32),
        pltpu.VMEM((C,), jnp.int32),
        pltpu.VMEM((C, D), jnp.float32),
        pltpu.VMEM((C, D), jnp.float32),
        pltpu.VMEM((8, D), jnp.float32),
        pltpu.VMEM_SHARED((AROWS, D), jnp.float32),
        pltpu.SemaphoreType.DMA,
        pltpu.SemaphoreType.DMA,
    ],
)
def _agg_kernel(u_hbm, src_hbm, dst_hbm, out_hbm,
                src_v, dst_v, adj_v, rows0, rows1, zbuf, agg_sh, sem0, sem1):
    cid = lax.axis_index("c")
    sid = lax.axis_index("s")
    tid = cid * NS + sid

    pltpu.sync_copy(src_hbm.at[tid], src_v)
    pltpu.sync_copy(dst_hbm.at[tid], dst_v)

    def _zfill(i, _):
        for k in range(D // 16):
            zbuf[i, pl.ds(k * 16, 16)] = _Z16()
        return 0

    lax.fori_loop(0, 8, _zfill, 0)

    def _issue(r, buf, sem):
        pltpu.async_copy(u_hbm.at[src_v.at[r]], buf, sem)

    def _wait(r, buf, sem):
        pltpu.make_async_copy(u_hbm.at[src_v.at[r]], buf, sem).wait()

    for half in range(2):
        base = half * HROWS

        def _zcopy(i, _):
            pltpu.sync_copy(zbuf, agg_sh.at[pl.ds(sid * ZPT + i * 8, 8)])
            return 0

        lax.fori_loop(0, ZPT // 8, _zcopy, 0)
        plsc.subcore_barrier()

        def _scatter(r, buf):
            for k in range(C // 16):
                d = dst_v[r, pl.ds(k * 16, 16)]
                adj = d - base
                inb = (adj >= 0) & (adj < HROWS)
                adj_v[pl.ds(k * 16, 16)] = jnp.where(
                    inb, adj, HROWS + (d & (TRASH - 1))
                )
            pltpu.sync_copy(buf, agg_sh.at[adj_v], add=True)

        _issue(0, rows0, sem0)
        _issue(1, rows1, sem1)

        def _body(j, _):
            a = 2 * j
            _wait(a, rows0, sem0)
            _scatter(a, rows0)
            _issue(a + 2, rows0, sem0)
            _wait(a + 1, rows1, sem1)
            _scatter(a + 1, rows1)
            _issue(a + 3, rows1, sem1)
            return 0

        lax.fori_loop(0, (CPT - 3) // 2, _body, 0)
        _wait(CPT - 3, rows0, sem0)
        _scatter(CPT - 3, rows0)
        _issue(CPT - 1, rows0, sem0)
        _wait(CPT - 2, rows1, sem1)
        _scatter(CPT - 2, rows1)
        _wait(CPT - 1, rows0, sem0)
        _scatter(CPT - 1, rows0)

        plsc.subcore_barrier()
        pltpu.sync_copy(
            agg_sh.at[pl.ds(sid * WPT, WPT)],
            out_hbm.at[cid, half, pl.ds(sid * WPT, WPT)],
        )
        plsc.subcore_barrier()


def _tc1_body(degp, x, w1, t_out, u_out, isd_out, sw_out):
    deg = degp[0, :] + degp[1, :] + 1.0
    isd = lax.rsqrt(deg)[:, None]
    sw = (1.0 / deg)[:, None]
    t = jnp.dot(x[...], w1[...], preferred_element_type=jnp.float32)
    t_out[...] = t
    u_out[...] = t * isd
    isd_out[...] = isd
    sw_out[...] = sw


def _tc1(degp, x, w1):
    return pl.pallas_call(
        _tc1_body,
        grid=(GRID,),
        in_specs=[
            pl.BlockSpec((NC, RB), lambda i: (0, i)),
            pl.BlockSpec((RB, D), lambda i: (i, 0)),
            pl.BlockSpec((D, D), lambda i: (0, 0)),
        ],
        out_specs=[
            pl.BlockSpec((RB, D), lambda i: (i, 0)),
            pl.BlockSpec((RB, D), lambda i: (i, 0)),
            pl.BlockSpec((RB, 1), lambda i: (i, 0)),
            pl.BlockSpec((RB, 1), lambda i: (i, 0)),
        ],
        out_shape=[
            jax.ShapeDtypeStruct((N, D), jnp.float32),
            jax.ShapeDtypeStruct((N, D), jnp.float32),
            jax.ShapeDtypeStruct((N, 1), jnp.float32),
            jax.ShapeDtypeStruct((N, 1), jnp.float32),
        ],
    )(degp, x, w1)


def _tc2_body(sp, tp, isd, sw, w, t_out, u_out):
    s = sp[0] + sp[1]
    h = jnp.maximum(isd[...] * s + sw[...] * tp[...], 0.0)
    t = jnp.dot(h, w[...], preferred_element_type=jnp.float32)
    t_out[...] = t
    u_out[...] = t * isd[...]


def _tc2(sp, tp, isd, sw, w):
    return pl.pallas_call(
        _tc2_body,
        grid=(GRID,),
        in_specs=[
            pl.BlockSpec((NC, RB, D), lambda i: (0, i, 0)),
            pl.BlockSpec((RB, D), lambda i: (i, 0)),
            pl.BlockSpec((RB, 1), lambda i: (i, 0)),
            pl.BlockSpec((RB, 1), lambda i: (i, 0)),
            pl.BlockSpec((D, D), lambda i: (0, 0)),
        ],
        out_specs=[
            pl.BlockSpec((RB, D), lambda i: (i, 0)),
            pl.BlockSpec((RB, D), lambda i: (i, 0)),
        ],
        out_shape=[
            jax.ShapeDtypeStruct((N, D), jnp.float32),
            jax.ShapeDtypeStruct((N, D), jnp.float32),
        ],
    )(sp, tp, isd, sw, w)


def _tc3_body(sp, tp, isd, sw, wd, out):
    i = pl.program_id(0)
    s = sp[0] + sp[1]
    h = jnp.maximum(isd[...] * s + sw[...] * tp[...], 0.0)
    t3 = jnp.maximum(
        jnp.dot(h, wd[...], preferred_element_type=jnp.float32), 0.0
    )
    rows = i * RB + lax.broadcasted_iota(jnp.int32, (RB, 1), 0)
    t3 = jnp.where(rows < N, t3, 0.0)
    part = jnp.sum(t3, axis=0, keepdims=True)

    @pl.when(i == 0)
    def _():
        out[...] = jnp.zeros_like(out)

    out[...] += part


def _tc3(sp, tp, isd, sw, wd):
    return pl.pallas_call(
        _tc3_body,
        grid=(GRID,),
        in_specs=[
            pl.BlockSpec((NC, RB, D), lambda i: (0, i, 0)),
            pl.BlockSpec((RB, D), lambda i: (i, 0)),
            pl.BlockSpec((RB, 1), lambda i: (i, 0)),
            pl.BlockSpec((RB, 1), lambda i: (i, 0)),
            pl.BlockSpec((D, OUTP), lambda i: (0, 0)),
        ],
        out_specs=pl.BlockSpec((1, OUTP), lambda i: (0, 0)),
        out_shape=jax.ShapeDtypeStruct((1, OUTP), jnp.float32),
    )(sp, tp, isd, sw, wd)


def kernel(x, edge_index, W1, W2, Wd):
    src = edge_index[0].reshape(NT, CPT, C)
    dst = edge_index[1].reshape(NT, CPT, C)
    degp = _degree_kernel(dst)
    t1, u1, isd, sw = _tc1(degp, x, W1)
    s1 = _agg_kernel(u1, src, dst).reshape(NC, NPAD, D)
    t2, u2 = _tc2(s1, t1, isd, sw, W2)
    s2 = _agg_kernel(u2, src, dst).reshape(NC, NPAD, D)
    wdp = jnp.pad(Wd, ((0, 0), (0, OUTP - OUT)))
    out = _tc3(s2, t2, isd, sw, wdp)
    return out.reshape(OUTP)[:OUT]


# speedup vs baseline: 25.3456x; 1.6520x over previous
import functools

import jax
import jax.numpy as jnp
from jax import lax
from jax.experimental import pallas as pl
from jax.experimental.pallas import tpu as pltpu
from jax.experimental.pallas import tpu_sc as plsc

N = 10000
E = 320000
D = 128
OUT = 51
OUTP = 64
NPAD = 10240
NC, NS = 2, 16
NT = NC * NS
C = 80
CPT = E // (NT * C)
RPT = NPAD // NS
ZROWS = 128
DSEG = NPAD // NS
RB = 1024
GRID = (N + RB - 1) // RB

_mesh = plsc.VectorSubcoreMesh(
    core_axis_name="c", subcore_axis_name="s", num_cores=NC, num_subcores=NS
)

_Z16 = functools.partial(jnp.zeros, (16,), jnp.float32)


@functools.partial(
    pl.kernel,
    out_type=jax.ShapeDtypeStruct((NC, NPAD), jnp.float32),
    mesh=_mesh,
    scratch_types=[
        pltpu.VMEM((CPT, C), jnp.int32),
        pltpu.VMEM((C,), jnp.float32),
        pltpu.VMEM((DSEG,), jnp.float32),
        pltpu.VMEM_SHARED((NPAD,), jnp.float32),
    ],
)
def _degree_kernel(dst_hbm, out_hbm, dst_v, ones_v, zbuf_v, deg_sh):
    cid = lax.axis_index("c")
    sid = lax.axis_index("s")
    tid = cid * NS + sid

    pltpu.sync_copy(dst_hbm.at[tid], dst_v)

    def _zfill(i, _):
        zbuf_v[pl.ds(i * 16, 16)] = _Z16()
        return 0

    lax.fori_loop(0, DSEG // 16, _zfill, 0)

    def _ofill(i, _):
        ones_v[pl.ds(i * 16, 16)] = jnp.ones((16,), jnp.float32)
        return 0

    lax.fori_loop(0, C // 16, _ofill, 0)

    pltpu.sync_copy(zbuf_v, deg_sh.at[pl.ds(sid * DSEG, DSEG)])
    plsc.subcore_barrier()

    def _scat(r, _):
        pltpu.sync_copy(ones_v, deg_sh.at[dst_v.at[r]], add=True)
        return 0

    lax.fori_loop(0, CPT, _scat, 0)
    plsc.subcore_barrier()

    pltpu.sync_copy(
        deg_sh.at[pl.ds(sid * DSEG, DSEG)],
        out_hbm.at[cid, pl.ds(sid * DSEG, DSEG)],
    )


@functools.partial(
    pl.kernel,
    out_type=jax.ShapeDtypeStruct((NC, NPAD, D), jnp.float32),
    mesh=_mesh,
    scratch_types=[
        pltpu.VMEM((4, 2, C), jnp.int32),
        pltpu.VMEM((C, D), jnp.float32),
        pltpu.VMEM((C, D), jnp.float32),
        pltpu.VMEM((8, D), jnp.float32),
        pltpu.VMEM_SHARED((NPAD, D), jnp.float32),
        pltpu.SemaphoreType.DMA,
        pltpu.SemaphoreType.DMA,
        pltpu.SemaphoreType.DMA,
        pltpu.SemaphoreType.DMA,
        pltpu.SemaphoreType.DMA,
        pltpu.SemaphoreType.DMA,
    ],
)
def _agg_kernel(u_hbm, edges_hbm, out_hbm,
                idx_v, rows0, rows1, zbuf, agg_sh,
                gsem0, gsem1, isem0, isem1, isem2, isem3):
    cid = lax.axis_index("c")
    sid = lax.axis_index("s")
    tid = cid * NS + sid

    gsems = (gsem0, gsem1)
    isems = (isem0, isem1, isem2, isem3)
    rbufs = (rows0, rows1)

    def _zfill(i, _):
        for k in range(D // 16):
            zbuf[i, pl.ds(k * 16, 16)] = _Z16()
        return 0

    lax.fori_loop(0, 8, _zfill, 0)

    def _zcopy(i, _):
        pltpu.sync_copy(zbuf, agg_sh.at[pl.ds(sid * RPT + i * 8, 8)])
        return 0

    lax.fori_loop(0, RPT // 8, _zcopy, 0)
    plsc.subcore_barrier()

    def _issue_idx(a, s):
        pltpu.async_copy(edges_hbm.at[tid, a], idx_v.at[s % 4], isems[s % 4])

    def _wait_idx(a, s):
        pltpu.make_async_copy(
            edges_hbm.at[tid, a], idx_v.at[s % 4], isems[s % 4]
        ).wait()

    def _issue_gather(s):
        pltpu.async_copy(
            u_hbm.at[idx_v.at[s % 4, 0]], rbufs[s % 2], gsems[s % 2]
        )

    def _wait_gather(s):
        pltpu.make_async_copy(
            u_hbm.at[idx_v.at[s % 4, 0]], rbufs[s % 2], gsems[s % 2]
        ).wait()

    def _scatter(s):
        pltpu.sync_copy(
            rbufs[s % 2], agg_sh.at[idx_v.at[s % 4, 1]], add=True
        )

    for a in range(4):
        _issue_idx(a, a)
    _wait_idx(0, 0)
    _issue_gather(0)
    _wait_idx(1, 1)
    _issue_gather(1)

    def _body(j, _):
        a4 = 4 * j
        for k in range(4):
            _wait_gather(k)
            _scatter(k)
            _wait_idx(a4 + k + 2, k + 2)
            _issue_gather(k + 2)
            _issue_idx(a4 + k + 4, k)
        return 0

    lax.fori_loop(0, (CPT - 5) // 4, _body, 0)
    base = CPT - 5
    _wait_gather(base)
    _scatter(base)
    _wait_idx(base + 2, base + 2)
    _issue_gather(base + 2)
    _issue_idx(base + 4, base)
    _wait_gather(base + 1)
    _scatter(base + 1)
    _wait_idx(base + 3, base + 3)
    _issue_gather(base + 3)
    _wait_gather(base + 2)
    _scatter(base + 2)
    _wait_idx(base + 4, base)
    _issue_gather(base + 4)
    _wait_gather(base + 3)
    _scatter(base + 3)
    _wait_gather(base + 4)
    _scatter(base + 4)

    plsc.subcore_barrier()
    pltpu.sync_copy(
        agg_sh.at[pl.ds(sid * RPT, RPT)],
        out_hbm.at[cid, pl.ds(sid * RPT, RPT)],
    )


def _tc1_body(degp, x, w1, t_out, u_out, isd_out, sw_out):
    deg = degp[0, :] + degp[1, :] + 1.0
    isd = lax.rsqrt(deg)[:, None]
    sw = (1.0 / deg)[:, None]
    t = jnp.dot(x[...], w1[...], preferred_element_type=jnp.float32)
    t_out[...] = t
    u_out[...] = t * isd
    isd_out[...] = isd
    sw_out[...] = sw


def _tc1(degp, x, w1):
    return pl.pallas_call(
        _tc1_body,
        grid=(GRID,),
        in_specs=[
            pl.BlockSpec((NC, RB), lambda i: (0, i)),
            pl.BlockSpec((RB, D), lambda i: (i, 0)),
            pl.BlockSpec((D, D), lambda i: (0, 0)),
        ],
        out_specs=[
            pl.BlockSpec((RB, D), lambda i: (i, 0)),
            pl.BlockSpec((RB, D), lambda i: (i, 0)),
            pl.BlockSpec((RB, 1), lambda i: (i, 0)),
            pl.BlockSpec((RB, 1), lambda i: (i, 0)),
        ],
        out_shape=[
            jax.ShapeDtypeStruct((N, D), jnp.float32),
            jax.ShapeDtypeStruct((N, D), jnp.float32),
            jax.ShapeDtypeStruct((N, 1), jnp.float32),
            jax.ShapeDtypeStruct((N, 1), jnp.float32),
        ],
    )(degp, x, w1)


def _tc2_body(sp, tp, isd, sw, w, t_out, u_out):
    s = sp[0] + sp[1]
    h = jnp.maximum(isd[...] * s + sw[...] * tp[...], 0.0)
    t = jnp.dot(h, w[...], preferred_element_type=jnp.float32)
    t_out[...] = t
    u_out[...] = t * isd[...]


def _tc2(sp, tp, isd, sw, w):
    return pl.pallas_call(
        _tc2_body,
        grid=(GRID,),
        in_specs=[
            pl.BlockSpec((NC, RB, D), lambda i: (0, i, 0)),
            pl.BlockSpec((RB, D), lambda i: (i, 0)),
            pl.BlockSpec((RB, 1), lambda i: (i, 0)),
            pl.BlockSpec((RB, 1), lambda i: (i, 0)),
            pl.BlockSpec((D, D), lambda i: (0, 0)),
        ],
        out_specs=[
            pl.BlockSpec((RB, D), lambda i: (i, 0)),
            pl.BlockSpec((RB, D), lambda i: (i, 0)),
        ],
        out_shape=[
            jax.ShapeDtypeStruct((N, D), jnp.float32),
            jax.ShapeDtypeStruct((N, D), jnp.float32),
        ],
    )(sp, tp, isd, sw, w)


def _tc3_body(sp, tp, isd, sw, wd, out):
    i = pl.program_id(0)
    s = sp[0] + sp[1]
    h = jnp.maximum(isd[...] * s + sw[...] * tp[...], 0.0)
    t3 = jnp.maximum(
        jnp.dot(h, wd[...], preferred_element_type=jnp.float32), 0.0
    )
    rows = i * RB + lax.broadcasted_iota(jnp.int32, (RB, 1), 0)
    t3 = jnp.where(rows < N, t3, 0.0)
    part = jnp.sum(t3, axis=0, keepdims=True)

    @pl.when(i == 0)
    def _():
        out[...] = jnp.zeros_like(out)

    out[...] += part


def _tc3(sp, tp, isd, sw, wd):
    return pl.pallas_call(
        _tc3_body,
        grid=(GRID,),
        in_specs=[
            pl.BlockSpec((NC, RB, D), lambda i: (0, i, 0)),
            pl.BlockSpec((RB, D), lambda i: (i, 0)),
            pl.BlockSpec((RB, 1), lambda i: (i, 0)),
            pl.BlockSpec((RB, 1), lambda i: (i, 0)),
            pl.BlockSpec((D, OUTP), lambda i: (0, 0)),
        ],
        out_specs=pl.BlockSpec((1, OUTP), lambda i: (0, 0)),
        out_shape=jax.ShapeDtypeStruct((1, OUTP), jnp.float32),
    )(sp, tp, isd, sw, wd)


def kernel(x, edge_index, W1, W2, Wd):
    dst = edge_index[1].reshape(NT, CPT, C)
    edges = edge_index.reshape(2, NT, CPT, C).transpose(1, 2, 0, 3)
    degp = _degree_kernel(dst)
    t1, u1, isd, sw = _tc1(degp, x, W1)
    s1 = _agg_kernel(u1, edges)
    t2, u2 = _tc2(s1, t1, isd, sw, W2)
    s2 = _agg_kernel(u2, edges)
    wdp = jnp.pad(Wd, ((0, 0), (0, OUTP - OUT)))
    out = _tc3(s2, t2, isd, sw, wdp)
    return out.reshape(OUTP)[:OUT]


# speedup vs baseline: 28.7239x; 1.1333x over previous
import functools

import jax
import jax.numpy as jnp
from jax import lax
from jax.experimental import pallas as pl
from jax.experimental.pallas import tpu as pltpu
from jax.experimental.pallas import tpu_sc as plsc

N = 10000
E = 320000
D = 128
OUT = 51
OUTP = 64
NPAD = 10240
NC, NS = 2, 16
NT = NC * NS
C = 80
CPT = E // (NT * C)
RPT = NPAD // NS
ZROWS = 128
DSEG = NPAD // NS
RB = 1024
GRID = (N + RB - 1) // RB

_mesh = plsc.VectorSubcoreMesh(
    core_axis_name="c", subcore_axis_name="s", num_cores=NC, num_subcores=NS
)

_Z16 = functools.partial(jnp.zeros, (16,), jnp.float32)


@functools.partial(
    pl.kernel,
    out_type=jax.ShapeDtypeStruct((NC, NPAD), jnp.float32),
    mesh=_mesh,
    scratch_types=[
        pltpu.VMEM((CPT, C), jnp.int32),
        pltpu.VMEM((C,), jnp.float32),
        pltpu.VMEM((DSEG,), jnp.float32),
        pltpu.VMEM_SHARED((NPAD,), jnp.float32),
        pltpu.SemaphoreType.DMA,
    ],
)
def _degree_kernel(dst_hbm, out_hbm, dst_v, ones_v, zbuf_v, deg_sh, dsem):
    cid = lax.axis_index("c")
    sid = lax.axis_index("s")
    tid = cid * NS + sid

    pltpu.sync_copy(dst_hbm.at[tid], dst_v)

    def _zfill(i, _):
        zbuf_v[pl.ds(i * 16, 16)] = _Z16()
        return 0

    lax.fori_loop(0, DSEG // 16, _zfill, 0)

    def _ofill(i, _):
        ones_v[pl.ds(i * 16, 16)] = jnp.ones((16,), jnp.float32)
        return 0

    lax.fori_loop(0, C // 16, _ofill, 0)

    pltpu.sync_copy(zbuf_v, deg_sh.at[pl.ds(sid * DSEG, DSEG)])
    plsc.subcore_barrier()

    def _scat(r, _):
        pltpu.async_copy(ones_v, deg_sh.at[dst_v.at[r]], dsem, add=True)
        return 0

    lax.fori_loop(0, CPT, _scat, 0)

    def _drain(r, _):
        pltpu.make_async_copy(ones_v, deg_sh.at[dst_v.at[r]], dsem).wait()
        return 0

    lax.fori_loop(0, CPT, _drain, 0)
    plsc.subcore_barrier()

    pltpu.sync_copy(
        deg_sh.at[pl.ds(sid * DSEG, DSEG)],
        out_hbm.at[cid, pl.ds(sid * DSEG, DSEG)],
    )


@functools.partial(
    pl.kernel,
    out_type=jax.ShapeDtypeStruct((NC, NPAD, D), jnp.float32),
    mesh=_mesh,
    scratch_types=[
        pltpu.VMEM((4, 2, C), jnp.int32),
        pltpu.VMEM((C, D), jnp.float32),
        pltpu.VMEM((C, D), jnp.float32),
        pltpu.VMEM((C, D), jnp.float32),
        pltpu.VMEM((8, D), jnp.float32),
        pltpu.VMEM_SHARED((NPAD, D), jnp.float32),
        pltpu.SemaphoreType.DMA,
        pltpu.SemaphoreType.DMA,
        pltpu.SemaphoreType.DMA,
        pltpu.SemaphoreType.DMA,
        pltpu.SemaphoreType.DMA,
        pltpu.SemaphoreType.DMA,
        pltpu.SemaphoreType.DMA,
        pltpu.SemaphoreType.DMA,
        pltpu.SemaphoreType.DMA,
        pltpu.SemaphoreType.DMA,
    ],
)
def _agg_kernel(u_hbm, edges_hbm, out_hbm,
                idx_v, rows0, rows1, rows2, zbuf, agg_sh,
                gsem0, gsem1, gsem2, ssem0, ssem1, ssem2,
                isem0, isem1, isem2, isem3):
    cid = lax.axis_index("c")
    sid = lax.axis_index("s")
    tid = cid * NS + sid

    gsems = (gsem0, gsem1, gsem2)
    ssems = (ssem0, ssem1, ssem2)
    isems = (isem0, isem1, isem2, isem3)
    rbufs = (rows0, rows1, rows2)

    def _zfill(i, _):
        for k in range(D // 16):
            zbuf[i, pl.ds(k * 16, 16)] = _Z16()
        return 0

    lax.fori_loop(0, 8, _zfill, 0)

    def _zcopy(i, _):
        pltpu.sync_copy(zbuf, agg_sh.at[pl.ds(sid * RPT + i * 8, 8)])
        return 0

    lax.fori_loop(0, RPT // 8, _zcopy, 0)
    plsc.subcore_barrier()

    def _issue_idx(a, s):
        pltpu.async_copy(edges_hbm.at[tid, a], idx_v.at[s % 4], isems[s % 4])

    def _wait_idx(a, s):
        pltpu.make_async_copy(
            edges_hbm.at[tid, a], idx_v.at[s % 4], isems[s % 4]
        ).wait()

    def _issue_gather(s):
        pltpu.async_copy(
            u_hbm.at[idx_v.at[s % 4, 0]], rbufs[s % 3], gsems[s % 3]
        )

    def _wait_gather(s):
        pltpu.make_async_copy(
            u_hbm.at[idx_v.at[s % 4, 0]], rbufs[s % 3], gsems[s % 3]
        ).wait()

    def _issue_scatter(s):
        pltpu.async_copy(
            rbufs[s % 3], agg_sh.at[idx_v.at[s % 4, 1]], ssems[s % 3],
            add=True,
        )

    def _wait_scatter(s):
        pltpu.make_async_copy(
            rbufs[s % 3], agg_sh.at[idx_v.at[s % 4, 1]], ssems[s % 3]
        ).wait()

    def _step(a, s, first=False, g2=True, i3=True):
        _wait_gather(s)
        _issue_scatter(s)
        if not first:
            _wait_scatter(s - 1)
        if g2:
            _wait_idx(a + 2, s + 2)
            _issue_gather(s + 2)
        if i3:
            _issue_idx(a + 3, s + 3)

    for a in range(3):
        _issue_idx(a, a)
    _wait_idx(0, 0)
    _issue_gather(0)
    _wait_idx(1, 1)
    _issue_gather(1)

    _step(0, 0, first=True)

    def _body(j, _):
        a12 = 12 * j
        for k in range(12):
            _step(a12 + k + 1, k + 1)
        return 0

    lax.fori_loop(0, (CPT - 5) // 12, _body, 0)
    _step(CPT - 4, CPT - 4)
    _step(CPT - 3, CPT - 3, i3=False)
    _step(CPT - 2, CPT - 2, g2=False, i3=False)
    _step(CPT - 1, CPT - 1, g2=False, i3=False)
    _wait_scatter(CPT - 1)

    plsc.subcore_barrier()
    pltpu.sync_copy(
        agg_sh.at[pl.ds(sid * RPT, RPT)],
        out_hbm.at[cid, pl.ds(sid * RPT, RPT)],
    )


def _tc1_body(degp, x, w1, t_out, u_out, isd_out, sw_out):
    deg = degp[0, :] + degp[1, :] + 1.0
    isd = lax.rsqrt(deg)[:, None]
    sw = (1.0 / deg)[:, None]
    t = jnp.dot(x[...], w1[...], preferred_element_type=jnp.float32)
    t_out[...] = t
    u_out[...] = t * isd
    isd_out[...] = isd
    sw_out[...] = sw


def _tc1(degp, x, w1):
    return pl.pallas_call(
        _tc1_body,
        grid=(GRID,),
        in_specs=[
            pl.BlockSpec((NC, RB), lambda i: (0, i)),
            pl.BlockSpec((RB, D), lambda i: (i, 0)),
            pl.BlockSpec((D, D), lambda i: (0, 0)),
        ],
        out_specs=[
            pl.BlockSpec((RB, D), lambda i: (i, 0)),
            pl.BlockSpec((RB, D), lambda i: (i, 0)),
            pl.BlockSpec((RB, 1), lambda i: (i, 0)),
            pl.BlockSpec((RB, 1), lambda i: (i, 0)),
        ],
        out_shape=[
            jax.ShapeDtypeStruct((N, D), jnp.float32),
            jax.ShapeDtypeStruct((N, D), jnp.float32),
            jax.ShapeDtypeStruct((N, 1), jnp.float32),
            jax.ShapeDtypeStruct((N, 1), jnp.float32),
        ],
    )(degp, x, w1)


def _tc2_body(sp, tp, isd, sw, w, t_out, u_out):
    s = sp[0] + sp[1]
    h = jnp.maximum(isd[...] * s + sw[...] * tp[...], 0.0)
    t = jnp.dot(h, w[...], preferred_element_type=jnp.float32)
    t_out[...] = t
    u_out[...] = t * isd[...]


def _tc2(sp, tp, isd, sw, w):
    return pl.pallas_call(
        _tc2_body,
        grid=(GRID,),
        in_specs=[
            pl.BlockSpec((NC, RB, D), lambda i: (0, i, 0)),
            pl.BlockSpec((RB, D), lambda i: (i, 0)),
            pl.BlockSpec((RB, 1), lambda i: (i, 0)),
            pl.BlockSpec((RB, 1), lambda i: (i, 0)),
            pl.BlockSpec((D, D), lambda i: (0, 0)),
        ],
        out_specs=[
            pl.BlockSpec((RB, D), lambda i: (i, 0)),
            pl.BlockSpec((RB, D), lambda i: (i, 0)),
        ],
        out_shape=[
            jax.ShapeDtypeStruct((N, D), jnp.float32),
            jax.ShapeDtypeStruct((N, D), jnp.float32),
        ],
    )(sp, tp, isd, sw, w)


def _tc3_body(sp, tp, isd, sw, wd, out):
    i = pl.program_id(0)
    s = sp[0] + sp[1]
    h = jnp.maximum(isd[...] * s + sw[...] * tp[...], 0.0)
    t3 = jnp.maximum(
        jnp.dot(h, wd[...], preferred_element_type=jnp.float32), 0.0
    )
    rows = i * RB + lax.broadcasted_iota(jnp.int32, (RB, 1), 0)
    t3 = jnp.where(rows < N, t3, 0.0)
    part = jnp.sum(t3, axis=0, keepdims=True)

    @pl.when(i == 0)
    def _():
        out[...] = jnp.zeros_like(out)

    out[...] += part


def _tc3(sp, tp, isd, sw, wd):
    return pl.pallas_call(
        _tc3_body,
        grid=(GRID,),
        in_specs=[
            pl.BlockSpec((NC, RB, D), lambda i: (0, i, 0)),
            pl.BlockSpec((RB, D), lambda i: (i, 0)),
            pl.BlockSpec((RB, 1), lambda i: (i, 0)),
            pl.BlockSpec((RB, 1), lambda i: (i, 0)),
            pl.BlockSpec((D, OUTP), lambda i: (0, 0)),
        ],
        out_specs=pl.BlockSpec((1, OUTP), lambda i: (0, 0)),
        out_shape=jax.ShapeDtypeStruct((1, OUTP), jnp.float32),
    )(sp, tp, isd, sw, wd)


def kernel(x, edge_index, W1, W2, Wd):
    dst = edge_index[1].reshape(NT, CPT, C)
    edges = edge_index.reshape(2, NT, CPT, C).transpose(1, 2, 0, 3)
    degp = _degree_kernel(dst)
    t1, u1, isd, sw = _tc1(degp, x, W1)
    s1 = _agg_kernel(u1, edges)
    t2, u2 = _tc2(s1, t1, isd, sw, W2)
    s2 = _agg_kernel(u2, edges)
    wdp = jnp.pad(Wd, ((0, 0), (0, OUTP - OUT)))
    out = _tc3(s2, t2, isd, sw, wdp)
    return out.reshape(OUTP)[:OUT]


# speedup vs baseline: 29.7064x; 1.0342x over previous
import functools

import jax
import jax.numpy as jnp
from jax import lax
from jax.experimental import pallas as pl
from jax.experimental.pallas import tpu as pltpu
from jax.experimental.pallas import tpu_sc as plsc

N = 10000
E = 320000
D = 128
OUT = 51
OUTP = 64
NPAD = 10240
NC, NS = 2, 16
NT = NC * NS
C = 80
CPT = E // (NT * C)
RPT = NPAD // NS
ZROWS = 128
DSEG = NPAD // NS
RB = 1024
GRID = (N + RB - 1) // RB

_mesh = plsc.VectorSubcoreMesh(
    core_axis_name="c", subcore_axis_name="s", num_cores=NC, num_subcores=NS
)

_Z16 = functools.partial(jnp.zeros, (16,), jnp.float32)


@functools.partial(
    pl.kernel,
    out_type=jax.ShapeDtypeStruct((NC, NPAD), jnp.float32),
    mesh=_mesh,
    scratch_types=[
        pltpu.VMEM((CPT, C), jnp.int32),
        pltpu.VMEM((C,), jnp.float32),
        pltpu.VMEM((DSEG,), jnp.float32),
        pltpu.VMEM_SHARED((NPAD,), jnp.float32),
        pltpu.SemaphoreType.DMA,
    ],
)
def _degree_kernel(dst_hbm, out_hbm, dst_v, ones_v, zbuf_v, deg_sh, dsem):
    cid = lax.axis_index("c")
    sid = lax.axis_index("s")
    tid = cid * NS + sid

    pltpu.sync_copy(dst_hbm.at[tid], dst_v)

    def _zfill(i, _):
        zbuf_v[pl.ds(i * 16, 16)] = _Z16()
        return 0

    lax.fori_loop(0, DSEG // 16, _zfill, 0)

    def _ofill(i, _):
        ones_v[pl.ds(i * 16, 16)] = jnp.ones((16,), jnp.float32)
        return 0

    lax.fori_loop(0, C // 16, _ofill, 0)

    pltpu.sync_copy(zbuf_v, deg_sh.at[pl.ds(sid * DSEG, DSEG)])
    plsc.subcore_barrier()

    def _scat(r, _):
        pltpu.async_copy(ones_v, deg_sh.at[dst_v.at[r]], dsem, add=True)
        return 0

    lax.fori_loop(0, CPT, _scat, 0)

    def _drain(r, _):
        pltpu.make_async_copy(ones_v, deg_sh.at[dst_v.at[r]], dsem).wait()
        return 0

    lax.fori_loop(0, CPT, _drain, 0)
    plsc.subcore_barrier()

    pltpu.sync_copy(
        deg_sh.at[pl.ds(sid * DSEG, DSEG)],
        out_hbm.at[cid, pl.ds(sid * DSEG, DSEG)],
    )


@functools.partial(
    pl.kernel,
    out_type=jax.ShapeDtypeStruct((NC, NPAD, D), jnp.float32),
    mesh=_mesh,
    scratch_types=[
        pltpu.VMEM((4, 2, C), jnp.int32),
        pltpu.VMEM((C, D), jnp.float32),
        pltpu.VMEM((C, D), jnp.float32),
        pltpu.VMEM((C, D), jnp.float32),
        pltpu.VMEM((8, D), jnp.float32),
        pltpu.VMEM_SHARED((NPAD, D), jnp.float32),
        pltpu.SemaphoreType.DMA,
        pltpu.SemaphoreType.DMA,
        pltpu.SemaphoreType.DMA,
        pltpu.SemaphoreType.DMA,
        pltpu.SemaphoreType.DMA,
        pltpu.SemaphoreType.DMA,
        pltpu.SemaphoreType.DMA,
        pltpu.SemaphoreType.DMA,
        pltpu.SemaphoreType.DMA,
        pltpu.SemaphoreType.DMA,
    ],
)
def _agg_kernel(u_hbm, edges_hbm, out_hbm,
                idx_v, rows0, rows1, rows2, zbuf, agg_sh,
                gsem0, gsem1, gsem2, ssem0, ssem1, ssem2,
                isem0, isem1, isem2, isem3):
    cid = lax.axis_index("c")
    sid = lax.axis_index("s")
    tid = cid * NS + sid

    gsems = (gsem0, gsem1, gsem2)
    ssems = (ssem0, ssem1, ssem2)
    isems = (isem0, isem1, isem2, isem3)
    rbufs = (rows0, rows1, rows2)

    def _zfill(i, _):
        for k in range(D // 16):
            zbuf[i, pl.ds(k * 16, 16)] = _Z16()
        return 0

    lax.fori_loop(0, 8, _zfill, 0)

    def _zcopy(i, _):
        pltpu.async_copy(
            zbuf, agg_sh.at[pl.ds(sid * RPT + i * 8, 8)], gsem0
        )
        return 0

    lax.fori_loop(0, RPT // 8, _zcopy, 0)

    def _zdrain(i, _):
        pltpu.make_async_copy(
            zbuf, agg_sh.at[pl.ds(sid * RPT + i * 8, 8)], gsem0
        ).wait()
        return 0

    lax.fori_loop(0, RPT // 8, _zdrain, 0)
    plsc.subcore_barrier()

    def _issue_idx(a, s):
        pltpu.async_copy(edges_hbm.at[tid, a], idx_v.at[s % 4], isems[s % 4])

    def _wait_idx(a, s):
        pltpu.make_async_copy(
            edges_hbm.at[tid, a], idx_v.at[s % 4], isems[s % 4]
        ).wait()

    def _issue_gather(s):
        pltpu.async_copy(
            u_hbm.at[idx_v.at[s % 4, 0]], rbufs[s % 3], gsems[s % 3]
        )

    def _wait_gather(s):
        pltpu.make_async_copy(
            u_hbm.at[idx_v.at[s % 4, 0]], rbufs[s % 3], gsems[s % 3]
        ).wait()

    def _issue_scatter(s):
        pltpu.async_copy(
            rbufs[s % 3], agg_sh.at[idx_v.at[s % 4, 1]], ssems[s % 3],
            add=True,
        )

    def _wait_scatter(s):
        pltpu.make_async_copy(
            rbufs[s % 3], agg_sh.at[idx_v.at[s % 4, 1]], ssems[s % 3]
        ).wait()

    def _step(a, s, first=False, g2=True, i3=True):
        _wait_gather(s)
        _issue_scatter(s)
        if not first:
            _wait_scatter(s - 1)
        if g2:
            _wait_idx(a + 2, s + 2)
            _issue_gather(s + 2)
        if i3:
            _issue_idx(a + 3, s + 3)

    for a in range(3):
        _issue_idx(a, a)
    _wait_idx(0, 0)
    _issue_gather(0)
    _wait_idx(1, 1)
    _issue_gather(1)

    _step(0, 0, first=True)

    def _body(j, _):
        a12 = 12 * j
        for k in range(12):
            _step(a12 + k + 1, k + 1)
        return 0

    lax.fori_loop(0, (CPT - 5) // 12, _body, 0)
    _step(CPT - 4, CPT - 4)
    _step(CPT - 3, CPT - 3, i3=False)
    _step(CPT - 2, CPT - 2, g2=False, i3=False)
    _step(CPT - 1, CPT - 1, g2=False, i3=False)
    _wait_scatter(CPT - 1)

    plsc.subcore_barrier()
    pltpu.sync_copy(
        agg_sh.at[pl.ds(sid * RPT, RPT)],
        out_hbm.at[cid, pl.ds(sid * RPT, RPT)],
    )


def _tc1_body(degp, x, w1, t_out, u_out, isd_out, sw_out):
    deg = degp[0, :] + degp[1, :] + 1.0
    isd = lax.rsqrt(deg)[:, None]
    sw = (1.0 / deg)[:, None]
    t = jnp.dot(x[...], w1[...], preferred_element_type=jnp.float32)
    t_out[...] = t
    u_out[...] = t * isd
    isd_out[...] = isd
    sw_out[...] = sw


def _tc1(degp, x, w1):
    return pl.pallas_call(
        _tc1_body,
        grid=(GRID,),
        in_specs=[
            pl.BlockSpec((NC, RB), lambda i: (0, i)),
            pl.BlockSpec((RB, D), lambda i: (i, 0)),
            pl.BlockSpec((D, D), lambda i: (0, 0)),
        ],
        out_specs=[
            pl.BlockSpec((RB, D), lambda i: (i, 0)),
            pl.BlockSpec((RB, D), lambda i: (i, 0)),
            pl.BlockSpec((RB, 1), lambda i: (i, 0)),
            pl.BlockSpec((RB, 1), lambda i: (i, 0)),
        ],
        out_shape=[
            jax.ShapeDtypeStruct((N, D), jnp.float32),
            jax.ShapeDtypeStruct((N, D), jnp.float32),
            jax.ShapeDtypeStruct((N, 1), jnp.float32),
            jax.ShapeDtypeStruct((N, 1), jnp.float32),
        ],
    )(degp, x, w1)


def _tc2_body(sp, tp, isd, sw, w, t_out, u_out):
    s = sp[0] + sp[1]
    h = jnp.maximum(isd[...] * s + sw[...] * tp[...], 0.0)
    t = jnp.dot(h, w[...], preferred_element_type=jnp.float32)
    t_out[...] = t
    u_out[...] = t * isd[...]


def _tc2(sp, tp, isd, sw, w):
    return pl.pallas_call(
        _tc2_body,
        grid=(GRID,),
        in_specs=[
            pl.BlockSpec((NC, RB, D), lambda i: (0, i, 0)),
            pl.BlockSpec((RB, D), lambda i: (i, 0)),
            pl.BlockSpec((RB, 1), lambda i: (i, 0)),
            pl.BlockSpec((RB, 1), lambda i: (i, 0)),
            pl.BlockSpec((D, D), lambda i: (0, 0)),
        ],
        out_specs=[
            pl.BlockSpec((RB, D), lambda i: (i, 0)),
            pl.BlockSpec((RB, D), lambda i: (i, 0)),
        ],
        out_shape=[
            jax.ShapeDtypeStruct((N, D), jnp.float32),
            jax.ShapeDtypeStruct((N, D), jnp.float32),
        ],
    )(sp, tp, isd, sw, w)


def _tc3_body(sp, tp, isd, sw, wd, out):
    i = pl.program_id(0)
    s = sp[0] + sp[1]
    h = jnp.maximum(isd[...] * s + sw[...] * tp[...], 0.0)
    t3 = jnp.maximum(
        jnp.dot(h, wd[...], preferred_element_type=jnp.float32), 0.0
    )
    rows = i * RB + lax.broadcasted_iota(jnp.int32, (RB, 1), 0)
    t3 = jnp.where(rows < N, t3, 0.0)
    part = jnp.sum(t3, axis=0, keepdims=True)

    @pl.when(i == 0)
    def _():
        out[...] = jnp.zeros_like(out)

    out[...] += part


def _tc3(sp, tp, isd, sw, wd):
    return pl.pallas_call(
        _tc3_body,
        grid=(GRID,),
        in_specs=[
            pl.BlockSpec((NC, RB, D), lambda i: (0, i, 0)),
            pl.BlockSpec((RB, D), lambda i: (i, 0)),
            pl.BlockSpec((RB, 1), lambda i: (i, 0)),
            pl.BlockSpec((RB, 1), lambda i: (i, 0)),
            pl.BlockSpec((D, OUTP), lambda i: (0, 0)),
        ],
        out_specs=pl.BlockSpec((1, OUTP), lambda i: (0, 0)),
        out_shape=jax.ShapeDtypeStruct((1, OUTP), jnp.float32),
    )(sp, tp, isd, sw, wd)


def kernel(x, edge_index, W1, W2, Wd):
    dst = edge_index[1].reshape(NT, CPT, C)
    edges = edge_index.reshape(2, NT, CPT, C).transpose(1, 2, 0, 3)
    degp = _degree_kernel(dst)
    t1, u1, isd, sw = _tc1(degp, x, W1)
    s1 = _agg_kernel(u1, edges)
    t2, u2 = _tc2(s1, t1, isd, sw, W2)
    s2 = _agg_kernel(u2, edges)
    wdp = jnp.pad(Wd, ((0, 0), (0, OUTP - OUT)))
    out = _tc3(s2, t2, isd, sw, wdp)
    return out.reshape(OUTP)[:OUT]
